# baseline (speedup 1.0000x reference)
import functools

import jax
import jax.numpy as jnp
from jax import lax
from jax.experimental import pallas as pl
from jax.experimental.pallas import tpu as pltpu
from jax.experimental.pallas import tpu_sc as plsc

NC = 2
NS = 16
LANES = 16
NW = NC * NS
BATCH = 16384
RANK = 32
BPW = BATCH // NW
IDX_CHUNK = 128
NCHUNK = BPW // IDX_CHUNK


def _pmf_body(uid_hbm, iid_hbm, ut_hbm, it_hbm, ub_hbm, ib_hbm, gb_hbm, lp_hbm,
              pred_hbm, unc_hbm, ue_hbm, ie_hbm,
              uid_v, iid_v, u_rows, i_rows, ub_v, ib_v, pred_v, gb_v, lp_v,
              sem, osem):
    c = lax.axis_index("c")
    s = lax.axis_index("s")
    wid = s * NC + c
    base = wid * BPW

    pltpu.sync_copy(uid_hbm.at[pl.ds(wid * NCHUNK, NCHUNK)], uid_v)
    pltpu.sync_copy(iid_hbm.at[pl.ds(wid * NCHUNK, NCHUNK)], iid_v)
    pltpu.sync_copy(gb_hbm, gb_v)
    pltpu.sync_copy(lp_hbm, lp_v)

    copies = []
    for j in range(NCHUNK):
        sl = pl.ds(j * IDX_CHUNK, IDX_CHUNK)
        copies.append(pltpu.async_copy(ut_hbm.at[uid_v.at[j]], u_rows.at[sl], sem))
        copies.append(pltpu.async_copy(it_hbm.at[iid_v.at[j]], i_rows.at[sl], sem))
        copies.append(pltpu.async_copy(ub_hbm.at[uid_v.at[j]], ub_v.at[sl], sem))
        copies.append(pltpu.async_copy(ib_hbm.at[iid_v.at[j]], ib_v.at[sl], sem))
    for cp in copies:
        cp.wait()

    out_u = pltpu.async_copy(u_rows, ue_hbm.at[pl.ds(base, BPW)], osem)
    out_i = pltpu.async_copy(i_rows, ie_hbm.at[pl.ds(base, BPW)], osem)

    gb = gb_v[...]
    lane = lax.iota(jnp.int32, LANES)

    def blk(b, carry):
        rows = lane + b * LANES
        acc = jnp.zeros((LANES,), jnp.float32)
        for r in range(RANK):
            colr = jnp.full((LANES,), r, jnp.int32)
            uc = plsc.load_gather(u_rows, [rows, colr])
            ic = plsc.load_gather(i_rows, [rows, colr])
            acc = acc + uc * ic
        off = pl.ds(b * LANES, LANES)
        pred_v[off] = acc + ub_v[off] + ib_v[off] + gb
        return carry

    lax.fori_loop(0, BPW // LANES, blk, 0)

    pltpu.sync_copy(pred_v, pred_hbm.at[pl.ds(base, BPW)])

    @pl.when(wid == 0)
    def _():
        lp_v[...] = 1.0 / jnp.exp(lp_v[...])
        pltpu.sync_copy(lp_v, unc_hbm)

    out_u.wait()
    out_i.wait()


@jax.jit
def kernel(user_ids, item_ids, user_table, item_table, user_bias, item_bias,
           global_bias, log_precision):
    uid = user_ids.astype(jnp.int32).reshape(NW * NCHUNK, IDX_CHUNK)
    iid = item_ids.astype(jnp.int32).reshape(NW * NCHUNK, IDX_CHUNK)
    ub = user_bias.reshape(-1)
    ib = item_bias.reshape(-1)
    gb = jnp.broadcast_to(global_bias.astype(jnp.float32), (LANES,))
    lp = jnp.broadcast_to(log_precision.astype(jnp.float32), (LANES,))

    mesh = plsc.VectorSubcoreMesh(core_axis_name="c", subcore_axis_name="s",
                                  num_cores=NC, num_subcores=NS)
    pred, unc, ue, ie = pl.kernel(
        _pmf_body,
        out_type=[
            jax.ShapeDtypeStruct((BATCH,), jnp.float32),
            jax.ShapeDtypeStruct((LANES,), jnp.float32),
            jax.ShapeDtypeStruct((BATCH, RANK), jnp.float32),
            jax.ShapeDtypeStruct((BATCH, RANK), jnp.float32),
        ],
        mesh=mesh,
        compiler_params=pltpu.CompilerParams(use_tc_tiling_on_sc=False,
                                             needs_layout_passes=False),
        scratch_types=[
            pltpu.VMEM((NCHUNK, IDX_CHUNK), jnp.int32),
            pltpu.VMEM((NCHUNK, IDX_CHUNK), jnp.int32),
            pltpu.VMEM((BPW, RANK), jnp.float32),
            pltpu.VMEM((BPW, RANK), jnp.float32),
            pltpu.VMEM((BPW,), jnp.float32),
            pltpu.VMEM((BPW,), jnp.float32),
            pltpu.VMEM((BPW,), jnp.float32),
            pltpu.VMEM((LANES,), jnp.float32),
            pltpu.VMEM((LANES,), jnp.float32),
            pltpu.SemaphoreType.DMA,
            pltpu.SemaphoreType.DMA,
        ],
    )(uid, iid, user_table, item_table, ub, ib, gb, lp)

    return (pred.reshape(BATCH, 1), unc[:1], ue, ie)
